# baseline (speedup 1.0000x reference)
import functools

import jax
import jax.numpy as jnp
from jax import lax
from jax.experimental import pallas as pl
import jax.experimental.pallas.tpu as pltpu
from jax.experimental.pallas import tpu_sc as plsc

SHAPE = (16, 32, 32)
D_PER = 80
EMBD = 240
NLANE = 16
NREG = D_PER // NLANE
HC = 2
NSUB = 16


def _sc_body(w0_hbm, w1_hbm, w2_hbm, out_hbm, bufs, spm, w0v, w1v, w2v, sems, dsems):
    T, H, W = SHAPE
    B = out_hbm.shape[0]
    BH = B // 2
    HH = H // 2
    n_rounds = HH // HC
    t = lax.axis_index("s")
    half = lax.axis_index("c")

    pltpu.sync_copy(w0_hbm.at[t], w0v)
    pltpu.sync_copy(w1_hbm.at[pl.ds(half * HH, HH)], w1v)
    pltpu.sync_copy(w2_hbm, w2v)

    w0regs = [w0v[pl.ds(k * NLANE, NLANE)] for k in range(NREG)]

    def drain(j):
        buf = bufs.at[j % 2]
        spo = spm.at[t, j % 2]
        dst_h = pl.ds(half * HH + j * HC, HC)
        for b in range(BH):
            pltpu.make_async_copy(
                buf, out_hbm.at[b, t, dst_h], sems.at[j % 2]
            ).wait()
        for b in range(BH, B):
            pltpu.make_async_copy(
                spo, out_hbm.at[b, t, dst_h], dsems.at[j % 2]
            ).wait()

    for j in range(n_rounds):
        buf = bufs.at[j % 2]
        spo = spm.at[t, j % 2]
        dst_h = pl.ds(half * HH + j * HC, HC)
        if j >= 2:
            drain(j - 2)

        w1regs = [
            [w1v[j * HC + h, pl.ds(k * NLANE, NLANE)] for k in range(NREG)]
            for h in range(HC)
        ]

        def w_body(w, _):
            for h in range(HC):
                for k in range(NREG):
                    buf[h, w, pl.ds(k * NLANE, NLANE)] = w0regs[k]
                for k in range(NREG):
                    buf[h, w, pl.ds(D_PER + k * NLANE, NLANE)] = w1regs[h][k]
                for k in range(NREG):
                    buf[h, w, pl.ds(2 * D_PER + k * NLANE, NLANE)] = w2v[
                        w, pl.ds(k * NLANE, NLANE)
                    ]
            return 0

        lax.fori_loop(0, W, w_body, 0)

        for b in range(BH):
            pltpu.make_async_copy(
                buf, out_hbm.at[b, t, dst_h], sems.at[j % 2]
            ).start()
        pltpu.sync_copy(buf, spo)
        for b in range(BH, B):
            pltpu.make_async_copy(
                spo, out_hbm.at[b, t, dst_h], dsems.at[j % 2]
            ).start()

    drain(n_rounds - 2)
    drain(n_rounds - 1)


def kernel(x, W0, W1, W2):
    B = x.shape[0]
    T, H, W = SHAPE
    HH = H // 2
    run = pl.kernel(
        _sc_body,
        out_type=jax.ShapeDtypeStruct((B, T, H, W, EMBD), jnp.float32),
        mesh=plsc.VectorSubcoreMesh(core_axis_name="c", subcore_axis_name="s"),
        scratch_types=[
            pltpu.VMEM((2, HC, W, EMBD), jnp.float32),
            pltpu.VMEM_SHARED((NSUB, 2, HC, W, EMBD), jnp.float32),
            pltpu.VMEM((D_PER,), jnp.float32),
            pltpu.VMEM((HH, D_PER), jnp.float32),
            pltpu.VMEM((W, D_PER), jnp.float32),
            pltpu.SemaphoreType.DMA((2,)),
            pltpu.SemaphoreType.DMA((2,)),
        ],
    )
    return run(W0, W1, W2)


# speedup vs baseline: 1.0323x; 1.0323x over previous
import jax
import jax.numpy as jnp
from jax import lax
from jax.experimental import pallas as pl
import jax.experimental.pallas.tpu as pltpu
from jax.experimental.pallas import tpu_sc as plsc

SHAPE = (16, 32, 32)
D_PER = 80
EMBD = 240
NLANE = 16
NREG = D_PER // NLANE
HC = 2


def _sc_body(w0_hbm, w1_hbm, w2_hbm, out_hbm, bufs, w0v, w1v, w2v, sems):
    T, H, W = SHAPE
    B = out_hbm.shape[0]
    HH = H // 2
    n_rounds = HH // HC
    t = lax.axis_index("s")
    half = lax.axis_index("c")

    pltpu.sync_copy(w0_hbm.at[t], w0v)
    pltpu.sync_copy(w1_hbm.at[pl.ds(half * HH, HH)], w1v)
    pltpu.sync_copy(w2_hbm, w2v)

    w0regs = [w0v[pl.ds(k * NLANE, NLANE)] for k in range(NREG)]

    for j in range(n_rounds):
        buf = bufs.at[j % 2]
        if j >= 2:
            for b in range(B):
                pltpu.make_async_copy(
                    buf,
                    out_hbm.at[b, t, pl.ds(half * HH + (j - 2) * HC, HC)],
                    sems.at[j % 2],
                ).wait()

        w1regs = [
            [w1v[j * HC + h, pl.ds(k * NLANE, NLANE)] for k in range(NREG)]
            for h in range(HC)
        ]

        def w_body(w, _):
            for h in range(HC):
                for k in range(NREG):
                    buf[h, w, pl.ds(k * NLANE, NLANE)] = w0regs[k]
                for k in range(NREG):
                    buf[h, w, pl.ds(D_PER + k * NLANE, NLANE)] = w1regs[h][k]
                for k in range(NREG):
                    buf[h, w, pl.ds(2 * D_PER + k * NLANE, NLANE)] = w2v[
                        w, pl.ds(k * NLANE, NLANE)
                    ]
            return 0

        lax.fori_loop(0, W, w_body, 0)

        for b in range(B):
            pltpu.make_async_copy(
                buf, out_hbm.at[b, t, pl.ds(half * HH + j * HC, HC)], sems.at[j % 2]
            ).start()

    for j in (n_rounds - 2, n_rounds - 1):
        for b in range(B):
            pltpu.make_async_copy(
                bufs.at[j % 2],
                out_hbm.at[b, t, pl.ds(half * HH + j * HC, HC)],
                sems.at[j % 2],
            ).wait()


def kernel(x, W0, W1, W2):
    B = x.shape[0]
    T, H, W = SHAPE
    HH = H // 2
    run = pl.kernel(
        _sc_body,
        out_type=jax.ShapeDtypeStruct((B, T, H, W, EMBD), jnp.float32),
        mesh=plsc.VectorSubcoreMesh(core_axis_name="c", subcore_axis_name="s"),
        scratch_types=[
            pltpu.VMEM((2, HC, W, EMBD), jnp.float32),
            pltpu.VMEM((D_PER,), jnp.float32),
            pltpu.VMEM((HH, D_PER), jnp.float32),
            pltpu.VMEM((W, D_PER), jnp.float32),
            pltpu.SemaphoreType.DMA((2,)),
        ],
    )
    return run(W0, W1, W2)
